# Initial kernel scaffold; baseline (speedup 1.0000x reference)
import jax
import jax.numpy as jnp
from jax.experimental import pallas as pl


def kernel(x, W, scale, per_expert_scale):
    raise NotImplementedError("write your pallas kernel here")


# speedup vs baseline: 12.6332x; 12.6332x over previous
import jax
import jax.numpy as jnp
from jax.experimental import pallas as pl

D_MODEL = 2816
N_EXPERTS = 128
TOP_K = 8
RMS_EPS = 1e-06
BLOCK_T = 512


def _router_block(x_ref, w_ref, scale_ref, pes_ref, out_ref):
    x = x_ref[...]
    v = jnp.mean(x * x, axis=-1, keepdims=True)
    h = x * jax.lax.rsqrt(v + RMS_EPS)
    h = h * (scale_ref[...] * (D_MODEL ** -0.5))
    scores = jax.lax.dot_general(
        h, w_ref[...], (((1,), (1,)), ((), ())),
        preferred_element_type=jnp.float32)

    col = jax.lax.broadcasted_iota(jnp.int32, scores.shape, 1)
    neg_inf = jnp.float32(-jnp.inf)
    mask = jnp.zeros(scores.shape, jnp.bool_)
    cur = scores
    row_max = jnp.max(scores, axis=-1, keepdims=True)
    for _ in range(TOP_K):
        m = jnp.max(cur, axis=-1, keepdims=True)
        is_m = cur == m
        cand = jnp.where(is_m, col, N_EXPERTS)
        sel = jnp.min(cand, axis=-1, keepdims=True)
        pick = col == sel
        mask = jnp.logical_or(mask, pick)
        cur = jnp.where(pick, neg_inf, cur)

    e = jnp.where(mask, jnp.exp(scores - row_max), 0.0)
    denom = jnp.sum(e, axis=-1, keepdims=True)
    out_ref[...] = (e / denom) * pes_ref[...]


def kernel(x, W, scale, per_expert_scale):
    B, S, D = x.shape
    T = B * S
    xf = x.reshape(T, D)
    scale2 = scale.reshape(1, D)
    pes2 = per_expert_scale.reshape(1, N_EXPERTS)
    grid = (T // BLOCK_T,)
    out = pl.pallas_call(
        _router_block,
        grid=grid,
        in_specs=[
            pl.BlockSpec((BLOCK_T, D), lambda i: (i, 0)),
            pl.BlockSpec((N_EXPERTS, D), lambda i: (0, 0)),
            pl.BlockSpec((1, D), lambda i: (0, 0)),
            pl.BlockSpec((1, N_EXPERTS), lambda i: (0, 0)),
        ],
        out_specs=pl.BlockSpec((BLOCK_T, N_EXPERTS), lambda i: (i, 0)),
        out_shape=jax.ShapeDtypeStruct((T, N_EXPERTS), jnp.float32),
    )(xf, W, scale2, pes2)
    return out.reshape(B, S, N_EXPERTS)


# speedup vs baseline: 14.3039x; 1.1323x over previous
import jax
import jax.numpy as jnp
from jax.experimental import pallas as pl

D_MODEL = 2816
N_EXPERTS = 128
TOP_K = 8
RMS_EPS = 1e-06
BLOCK_T = 512

_DN_CONTRACT_LAST = (((1,), (1,)), ((), ()))


def _router_block(x_ref, ws_ref, ones_ref, pes_ref, out_ref):
    x = x_ref[...]
    st = jax.lax.dot_general(ws_ref[...], x, _DN_CONTRACT_LAST,
                             preferred_element_type=jnp.float32)
    ssq = jax.lax.dot_general(ones_ref[...], x * x, _DN_CONTRACT_LAST,
                              preferred_element_type=jnp.float32)
    r = jax.lax.rsqrt(ssq * (1.0 / D_MODEL) + RMS_EPS)
    scores = st * r

    row = jax.lax.broadcasted_iota(jnp.int32, scores.shape, 0)
    neg_inf = jnp.float32(-jnp.inf)
    mask = jnp.zeros(scores.shape, jnp.bool_)
    cur = scores
    smax = jnp.max(scores, axis=0, keepdims=True)
    for _ in range(TOP_K):
        m = jnp.max(cur, axis=0, keepdims=True)
        is_m = cur == m
        cand = jnp.where(is_m, row, N_EXPERTS)
        sel = jnp.min(cand, axis=0, keepdims=True)
        pick = row == sel
        mask = jnp.logical_or(mask, pick)
        cur = jnp.where(pick, neg_inf, cur)

    e = jnp.where(mask, jnp.exp(scores - smax), 0.0)
    denom = jnp.sum(e, axis=0, keepdims=True)
    outt = e * (1.0 / denom) * pes_ref[...]
    out_ref[...] = outt.T


def kernel(x, W, scale, per_expert_scale):
    B, S, D = x.shape
    T = B * S
    xf = x.reshape(T, D)
    ws = W * (scale * (D_MODEL ** -0.5))
    ones_row = jnp.ones((1, D), jnp.float32)
    pes2 = per_expert_scale.reshape(N_EXPERTS, 1)
    grid = (T // BLOCK_T,)
    out = pl.pallas_call(
        _router_block,
        grid=grid,
        in_specs=[
            pl.BlockSpec((BLOCK_T, D), lambda i: (i, 0)),
            pl.BlockSpec((N_EXPERTS, D), lambda i: (0, 0)),
            pl.BlockSpec((1, D), lambda i: (0, 0)),
            pl.BlockSpec((N_EXPERTS, 1), lambda i: (0, 0)),
        ],
        out_specs=pl.BlockSpec((BLOCK_T, N_EXPERTS), lambda i: (i, 0)),
        out_shape=jax.ShapeDtypeStruct((T, N_EXPERTS), jnp.float32),
    )(xf, ws, ones_row, pes2)
    return out.reshape(B, S, N_EXPERTS)


# speedup vs baseline: 16.8811x; 1.1802x over previous
import jax
import jax.numpy as jnp
from jax.experimental import pallas as pl

D_MODEL = 2816
N_EXPERTS = 128
TOP_K = 8
RMS_EPS = 1e-06
BLOCK_T = 512

_DN_CONTRACT_LAST = (((1,), (1,)), ((), ()))


def _router_block(x_ref, ws_ref, ones_ref, pes_ref, out_ref):
    x = x_ref[...]
    z = jax.lax.dot_general(x, ws_ref[...], _DN_CONTRACT_LAST,
                            preferred_element_type=jnp.float32)
    ssq = jax.lax.dot_general(x * x, ones_ref[...], _DN_CONTRACT_LAST,
                              preferred_element_type=jnp.float32)
    r = jax.lax.rsqrt(ssq * (1.0 / D_MODEL) + RMS_EPS)
    scores = z.T * r.T

    row = jax.lax.broadcasted_iota(jnp.int32, scores.shape, 0)
    neg_inf = jnp.float32(-jnp.inf)
    mask = jnp.zeros(scores.shape, jnp.bool_)
    cur = scores
    smax = jnp.max(scores, axis=0, keepdims=True)
    for _ in range(TOP_K):
        m = jnp.max(cur, axis=0, keepdims=True)
        is_m = cur == m
        cand = jnp.where(is_m, row, N_EXPERTS)
        sel = jnp.min(cand, axis=0, keepdims=True)
        pick = row == sel
        mask = jnp.logical_or(mask, pick)
        cur = jnp.where(pick, neg_inf, cur)

    e = jnp.where(mask, jnp.exp(scores - smax), 0.0)
    denom = jnp.sum(e, axis=0, keepdims=True)
    outt = e * (1.0 / denom) * pes_ref[...]
    out_ref[...] = outt.T


def kernel(x, W, scale, per_expert_scale):
    B, S, D = x.shape
    T = B * S
    xf = x.reshape(T, D)
    ws = W * (scale * (D_MODEL ** -0.5))
    ones_row = jnp.ones((1, D), jnp.float32)
    pes2 = per_expert_scale.reshape(N_EXPERTS, 1)
    grid = (T // BLOCK_T,)
    out = pl.pallas_call(
        _router_block,
        grid=grid,
        in_specs=[
            pl.BlockSpec((BLOCK_T, D), lambda i: (i, 0)),
            pl.BlockSpec((N_EXPERTS, D), lambda i: (0, 0)),
            pl.BlockSpec((1, D), lambda i: (0, 0)),
            pl.BlockSpec((N_EXPERTS, 1), lambda i: (0, 0)),
        ],
        out_specs=pl.BlockSpec((BLOCK_T, N_EXPERTS), lambda i: (i, 0)),
        out_shape=jax.ShapeDtypeStruct((T, N_EXPERTS), jnp.float32),
    )(xf, ws, ones_row, pes2)
    return out.reshape(B, S, N_EXPERTS)
